# baseline (speedup 1.0000x reference)
import functools
import math

import jax
import jax.numpy as jnp
from jax import lax
from jax.experimental import pallas as pl
from jax.experimental.pallas import tpu as pltpu
from jax.experimental.pallas import tpu_sc as plsc

ALPHA = 0.99
EPS = 1e-06
LOG_ALPHA = math.log(ALPHA)
M = 100000
D = 64
B = 16384

NC = 2
NS = 16
NW = NC * NS
FPW = D // NW
BROWS = B // 128
P0R = BROWS // NS
NBCH = 8
CROWS = BROWS // NBCH
CB = CROWS * 128

_mesh = plsc.VectorSubcoreMesh(core_axis_name="c", subcore_axis_name="s")


@functools.partial(
    pl.kernel,
    mesh=_mesh,
    out_type=jax.ShapeDtypeStruct((D, B), jnp.float32),
    compiler_params=pltpu.CompilerParams(needs_layout_passes=False),
    scratch_types=[
        pltpu.VMEM((P0R, 128), jnp.int32),
        pltpu.VMEM((P0R, 128), jnp.float32),
        pltpu.VMEM((P0R, 128), jnp.float32),
        pltpu.VMEM_SHARED((BROWS, 128), jnp.float32),
        pltpu.VMEM((M,), jnp.float32),
        pltpu.VMEM((2, CROWS, 128), jnp.int32),
        pltpu.VMEM((BROWS, 128), jnp.float32),
        pltpu.VMEM((2, CB), jnp.float32),
        pltpu.SemaphoreType.DMA,
        pltpu.SemaphoreType.DMA,
        pltpu.SemaphoreType.DMA,
        pltpu.SemaphoreType.DMA,
        pltpu.SemaphoreType.DMA,
    ],
)
def _ema_sc(i_hbm, ct_hbm, counts_hbm, out_hbm, idx0_v, cnt0_v, scale0_v,
            scale_sh, row_v, idxc_v, scalef_v, outc_v, sem_g, sem_w,
            sem_r, sem_p0, sem_p1):
    cid = lax.axis_index("c")
    sid = lax.axis_index("s")
    wid = sid * NC + cid

    row_cp = pltpu.async_copy(ct_hbm.at[wid * FPW], row_v, sem_r)

    pltpu.sync_copy(i_hbm.at[pl.ds(sid * P0R, P0R)], idx0_v)
    cnt_copies = [
        pltpu.async_copy(counts_hbm.at[idx0_v.at[r]], cnt0_v.at[r], sem_g)
        for r in range(P0R)
    ]
    for c in cnt_copies:
        c.wait()
    for r in range(P0R):
        for c in range(128 // 16):
            sl = pl.ds(c * 16, 16)
            scale0_v[r, sl] = 1.0 / (
                1.0 + EPS - jnp.exp(cnt0_v[r, sl] * LOG_ALPHA))
    pltpu.sync_copy(scale0_v, scale_sh.at[pl.ds(sid * P0R, P0R)])
    plsc.subcore_barrier()
    pltpu.sync_copy(scale_sh, scalef_v)

    psems = (sem_p0, sem_p1)

    def prefetch(s):
        k = s % NBCH
        pb = s % 2
        return [
            pltpu.async_copy(
                i_hbm.at[pl.ds(k * CROWS, CROWS)], idxc_v.at[pb],
                psems[pb]),
        ]

    nsteps = FPW * NBCH
    pending_p = {0: prefetch(0)}
    pending_w = {}
    for s in range(nsteps):
        f, k = s // NBCH, s % NBCH
        if k == 0:
            if f == 0:
                row_cp.wait()
            else:
                pltpu.sync_copy(ct_hbm.at[wid * FPW + f], row_v)
        for c in pending_p.pop(s):
            c.wait()
        if s + 1 < nsteps:
            pending_p[s + 1] = prefetch(s + 1)
        if s - 2 in pending_w:
            pending_w.pop(s - 2).wait()
        bb = s % 2
        d = wid * FPW + f

        def body(rr, carry):
            sls = [pl.ds(c * 16, 16) for c in range(128 // 16)]
            idxs = [idxc_v[bb, rr, sl] for sl in sls]
            gath = [plsc.load_gather(row_v, [ix]) for ix in idxs]
            scls = [scalef_v[k * CROWS + rr, sl] for sl in sls]
            for c in range(128 // 16):
                outc_v[bb, pl.ds(rr * 128 + c * 16, 16)] = (
                    gath[c] * scls[c])
            return carry

        lax.fori_loop(0, CROWS, body, 0)
        pending_w[s] = pltpu.async_copy(
            outc_v.at[bb], out_hbm.at[d, pl.ds(k * CB, CB)], sem_w)
    for key in sorted(pending_w):
        pending_w.pop(key).wait()


def kernel(i, x, centers, counts):
    del x
    i2d = i.astype(jnp.int32).reshape(BROWS, 128)
    out_t = _ema_sc(i2d, centers.T, counts)
    return out_t.T


# speedup vs baseline: 1.0904x; 1.0904x over previous
import functools
import math

import jax
import jax.numpy as jnp
from jax import lax
from jax.experimental import pallas as pl
from jax.experimental.pallas import tpu as pltpu
from jax.experimental.pallas import tpu_sc as plsc

ALPHA = 0.99
EPS = 1e-06
LOG_ALPHA = math.log(ALPHA)
M = 100000
D = 64
B = 16384

NC = 2
NS = 16
NW = NC * NS
FPW = D // NW
BROWS = B // 128
P0R = BROWS // NS
NBCH = 4
CROWS = BROWS // NBCH
CB = CROWS * 128

_mesh = plsc.VectorSubcoreMesh(core_axis_name="c", subcore_axis_name="s")


@functools.partial(
    pl.kernel,
    mesh=_mesh,
    out_type=(jax.ShapeDtypeStruct((D, B), jnp.float32),
              jax.ShapeDtypeStruct((NC, BROWS, 128), jnp.float32)),
    compiler_params=pltpu.CompilerParams(needs_layout_passes=False),
    scratch_types=[
        pltpu.VMEM((P0R, 128), jnp.int32),
        pltpu.VMEM((P0R, 128), jnp.float32),
        pltpu.VMEM((P0R, 128), jnp.float32),
        pltpu.VMEM((M,), jnp.float32),
        pltpu.VMEM((2, CROWS, 128), jnp.int32),
        pltpu.VMEM((2, CROWS, 128), jnp.float32),
        pltpu.VMEM((2, CB), jnp.float32),
        pltpu.SemaphoreType.DMA,
        pltpu.SemaphoreType.DMA,
        pltpu.SemaphoreType.DMA,
        pltpu.SemaphoreType.DMA,
        pltpu.SemaphoreType.DMA,
    ],
)
def _ema_sc(i_hbm, ct_hbm, counts_hbm, out_hbm, scale_hbm, idx0_v,
            cnt0_v, scale0_v, row_v, idxc_v, scalec_v, outc_v, sem_g,
            sem_w, sem_r, sem_p0, sem_p1):
    cid = lax.axis_index("c")
    sid = lax.axis_index("s")
    wid = sid * NC + cid

    row_cp = pltpu.async_copy(ct_hbm.at[wid * FPW], row_v, sem_r)

    pltpu.sync_copy(i_hbm.at[pl.ds(sid * P0R, P0R)], idx0_v)
    cnt_copies = [
        pltpu.async_copy(counts_hbm.at[idx0_v.at[r]], cnt0_v.at[r], sem_g)
        for r in range(P0R)
    ]
    for c in cnt_copies:
        c.wait()
    for r in range(P0R):
        for c in range(128 // 16):
            sl = pl.ds(c * 16, 16)
            scale0_v[r, sl] = 1.0 / (
                1.0 + EPS - jnp.exp(cnt0_v[r, sl] * LOG_ALPHA))
    pltpu.sync_copy(scale0_v, scale_hbm.at[cid, pl.ds(sid * P0R, P0R)])
    plsc.subcore_barrier()

    psems = (sem_p0, sem_p1)

    def prefetch(s):
        k = s % NBCH
        pb = s % 2
        return [
            pltpu.async_copy(
                i_hbm.at[pl.ds(k * CROWS, CROWS)], idxc_v.at[pb],
                psems[pb]),
            pltpu.async_copy(
                scale_hbm.at[cid, pl.ds(k * CROWS, CROWS)],
                scalec_v.at[pb], psems[pb]),
        ]

    nsteps = FPW * NBCH
    pending_p = {0: prefetch(0)}
    pending_w = {}
    for s in range(nsteps):
        f, k = s // NBCH, s % NBCH
        if k == 0:
            if f == 0:
                row_cp.wait()
            else:
                pltpu.sync_copy(ct_hbm.at[wid * FPW + f], row_v)
        for c in pending_p.pop(s):
            c.wait()
        if s + 1 < nsteps:
            pending_p[s + 1] = prefetch(s + 1)
        if s - 2 in pending_w:
            pending_w.pop(s - 2).wait()
        bb = s % 2
        d = wid * FPW + f

        def body(rr, carry):
            sls = [pl.ds(c * 16, 16) for c in range(128 // 16)]
            idxs = [idxc_v[bb, rr, sl] for sl in sls]
            gath = [plsc.load_gather(row_v, [ix]) for ix in idxs]
            scls = [scalec_v[bb, rr, sl] for sl in sls]
            for c in range(128 // 16):
                outc_v[bb, pl.ds(rr * 128 + c * 16, 16)] = (
                    gath[c] * scls[c])
            return carry

        lax.fori_loop(0, CROWS, body, 0)
        pending_w[s] = pltpu.async_copy(
            outc_v.at[bb], out_hbm.at[d, pl.ds(k * CB, CB)], sem_w)
    for key in sorted(pending_w):
        pending_w.pop(key).wait()


def kernel(i, x, centers, counts):
    del x
    i2d = i.astype(jnp.int32).reshape(BROWS, 128)
    out_t, _ = _ema_sc(i2d, centers.T, counts)
    return out_t.T


# speedup vs baseline: 1.1574x; 1.0615x over previous
import functools
import math

import jax
import jax.numpy as jnp
from jax import lax
from jax.experimental import pallas as pl
from jax.experimental.pallas import tpu as pltpu
from jax.experimental.pallas import tpu_sc as plsc

ALPHA = 0.99
EPS = 1e-06
LOG_ALPHA = math.log(ALPHA)
M = 100000
D = 64
B = 16384

NC = 2
NS = 16
NW = NC * NS
FPW = D // NW
BROWS = B // 128
P0R = BROWS // NS
NBCH = 4
CROWS = BROWS // NBCH
CB = CROWS * 128

_mesh = plsc.VectorSubcoreMesh(core_axis_name="c", subcore_axis_name="s")


@functools.partial(
    pl.kernel,
    mesh=_mesh,
    out_type=jax.ShapeDtypeStruct((D, B), jnp.float32),
    compiler_params=pltpu.CompilerParams(needs_layout_passes=False),
    scratch_types=[
        pltpu.VMEM_SHARED((BROWS, 128), jnp.float32),
        pltpu.VMEM((M,), jnp.float32),
        pltpu.VMEM((BROWS, 128), jnp.int32),
        pltpu.VMEM((CROWS, 128), jnp.float32),
        pltpu.VMEM((2, CB), jnp.float32),
        pltpu.SemaphoreType.DMA,
        pltpu.SemaphoreType.DMA,
        pltpu.SemaphoreType.DMA,
    ],
)
def _ema_sc(i_hbm, ct_hbm, counts_hbm, out_hbm, scale_sh, row_v, idxf_v,
            scalec_v, outc_v, sem_g, sem_w, sem_r):
    cid = lax.axis_index("c")
    sid = lax.axis_index("s")
    wid = sid * NC + cid

    row_cp = pltpu.async_copy(ct_hbm.at[wid * FPW], row_v, sem_r)

    pltpu.sync_copy(i_hbm, idxf_v)
    cnt_copies = [
        pltpu.async_copy(counts_hbm.at[idxf_v.at[sid * P0R + r]],
                         scalec_v.at[r], sem_g)
        for r in range(P0R)
    ]
    for c in cnt_copies:
        c.wait()
    for r in range(P0R):
        for c in range(128 // 16):
            sl = pl.ds(c * 16, 16)
            scalec_v[P0R + r, sl] = 1.0 / (
                1.0 + EPS - jnp.exp(scalec_v[r, sl] * LOG_ALPHA))
    pltpu.sync_copy(scalec_v.at[pl.ds(P0R, P0R)],
                    scale_sh.at[pl.ds(sid * P0R, P0R)])
    plsc.subcore_barrier()

    nsteps = FPW * NBCH
    pending_w = {}
    for s in range(nsteps):
        f, k = s // NBCH, s % NBCH
        if k == 0:
            if f == 0:
                row_cp.wait()
            else:
                pltpu.sync_copy(ct_hbm.at[wid * FPW + f], row_v)
        pltpu.sync_copy(scale_sh.at[pl.ds(k * CROWS, CROWS)], scalec_v)
        if s - 2 in pending_w:
            pending_w.pop(s - 2).wait()
        bb = s % 2
        d = wid * FPW + f

        def body(rr, carry):
            sls = [pl.ds(c * 16, 16) for c in range(128 // 16)]
            idxs = [idxf_v[k * CROWS + rr, sl] for sl in sls]
            gath = [plsc.load_gather(row_v, [ix]) for ix in idxs]
            scls = [scalec_v[rr, sl] for sl in sls]
            for c in range(128 // 16):
                outc_v[bb, pl.ds(rr * 128 + c * 16, 16)] = (
                    gath[c] * scls[c])
            return carry

        lax.fori_loop(0, CROWS, body, 0)
        pending_w[s] = pltpu.async_copy(
            outc_v.at[bb], out_hbm.at[d, pl.ds(k * CB, CB)], sem_w)
    for key in sorted(pending_w):
        pending_w.pop(key).wait()


def kernel(i, x, centers, counts):
    del x
    i2d = i.astype(jnp.int32).reshape(BROWS, 128)
    out_t = _ema_sc(i2d, centers.T, counts)
    return out_t.T
